# Initial kernel scaffold; baseline (speedup 1.0000x reference)
import jax
import jax.numpy as jnp
from jax.experimental import pallas as pl


def kernel(x, Wg, bg, Wa, ba, W1, b1, W2, b2):
    raise NotImplementedError("write your pallas kernel here")


# speedup vs baseline: 2.3857x; 2.3857x over previous
import functools

import jax
import jax.numpy as jnp
from jax.experimental import pallas as pl
from jax.experimental.pallas import tpu as pltpu

E = 8
K = 2
D = 768
H = 1536

BT = 512
HC = 768


def _moe_body(x_ref, Wg_ref, bg_ref, Wa_ref, ba_ref, W1_ref, b1_ref,
              W2_ref, b2_ref, out_ref, coef_ref):
    e = pl.program_id(1)
    hc = pl.program_id(2)

    @pl.when(jnp.logical_and(e == 0, hc == 0))
    def _gating():
        xb = x_ref[...]
        logits = jnp.dot(xb, Wg_ref[...],
                         preferred_element_type=jnp.float32) + bg_ref[...]
        idx = jax.lax.broadcasted_iota(jnp.int32, logits.shape, 1)
        m1 = jnp.max(logits, axis=-1, keepdims=True)
        a1 = jnp.min(jnp.where(logits == m1, idx, E), axis=-1, keepdims=True)
        l2 = jnp.where(idx == a1, -jnp.inf, logits)
        m2 = jnp.max(l2, axis=-1, keepdims=True)
        a2 = jnp.min(jnp.where(l2 == m2, idx, E), axis=-1, keepdims=True)
        e2 = jnp.exp(m2 - m1)
        s = 1.0 + e2
        w1 = 1.0 / s
        w2 = e2 / s
        coef_ref[...] = jnp.where(idx == a1, w1,
                                  jnp.where(idx == a2, w2, 0.0))
        out_ref[...] = jnp.zeros_like(out_ref)

    xb = x_ref[...]
    ha = jnp.dot(xb, Wa_ref[0], preferred_element_type=jnp.float32) + ba_ref[0]
    ha = ha * jax.nn.sigmoid(ha)
    h1 = jnp.dot(xb, W1_ref[0], preferred_element_type=jnp.float32) + b1_ref[0]
    h = ha * h1
    y = jnp.dot(h, W2_ref[0], preferred_element_type=jnp.float32)

    idx = jax.lax.broadcasted_iota(jnp.int32, coef_ref.shape, 1)
    c = jnp.sum(coef_ref[...] * (idx == e), axis=-1, keepdims=True)

    @pl.when(hc == 0)
    def _add_bias():
        out_ref[...] += c * (y + b2_ref[0])

    @pl.when(hc != 0)
    def _no_bias():
        out_ref[...] += c * y


@functools.partial(jax.jit, static_argnames=())
def kernel(x, Wg, bg, Wa, ba, W1, b1, W2, b2):
    B, S, _ = x.shape
    T = B * S
    xf = x.reshape(T, D)
    grid = (T // BT, E, H // HC)
    out = pl.pallas_call(
        _moe_body,
        grid=grid,
        in_specs=[
            pl.BlockSpec((BT, D), lambda i, e, h: (i, 0)),
            pl.BlockSpec((D, E), lambda i, e, h: (0, 0)),
            pl.BlockSpec((1, E), lambda i, e, h: (0, 0)),
            pl.BlockSpec((1, D, HC), lambda i, e, h: (e, 0, h)),
            pl.BlockSpec((1, 1, HC), lambda i, e, h: (e, 0, h)),
            pl.BlockSpec((1, D, HC), lambda i, e, h: (e, 0, h)),
            pl.BlockSpec((1, 1, HC), lambda i, e, h: (e, 0, h)),
            pl.BlockSpec((1, HC, D), lambda i, e, h: (e, h, 0)),
            pl.BlockSpec((1, 1, D), lambda i, e, h: (e, 0, 0)),
        ],
        out_specs=pl.BlockSpec((BT, D), lambda i, e, h: (i, 0)),
        out_shape=jax.ShapeDtypeStruct((T, D), jnp.float32),
        scratch_shapes=[pltpu.VMEM((BT, E), jnp.float32)],
        compiler_params=pltpu.CompilerParams(
            dimension_semantics=("parallel", "arbitrary", "arbitrary"),
        ),
    )(xf, Wg, bg.reshape(1, E), Wa, ba.reshape(E, 1, H), W1,
      b1.reshape(E, 1, H), W2, b2.reshape(E, 1, D))
    return out.reshape(B, S, D)
